# baseline (speedup 1.0000x reference)
import functools

import jax
import jax.numpy as jnp
import numpy as np
from jax import lax
from jax.experimental import pallas as pl
from jax.experimental.pallas import tpu as pltpu
from jax.experimental.pallas import tpu_sc as plsc

LANES = 16
SUB = 80
CW = 16
NC = 2
NS = 16


def _feature_perm(hd):
    blk = np.arange(LANES)
    p = np.empty(hd, np.int32)
    for j in range(hd // (2 * LANES)):
        p[32 * j + 2 * blk] = 32 * j + blk
        p[32 * j + 2 * blk + 1] = 32 * j + LANES + blk
    return p


def _mm_body(h_ref, w_ref, o_ref):
    o_ref[...] = jnp.dot(h_ref[...], w_ref[...],
                         preferred_element_type=jnp.float32
                         ).astype(jnp.bfloat16)


def _matmul(h, w, bm):
    n, hd = h.shape
    return pl.pallas_call(
        _mm_body,
        grid=(n // bm,),
        in_specs=[
            pl.BlockSpec((bm, hd), lambda i: (i, 0)),
            pl.BlockSpec((hd, hd), lambda i: (0, 0)),
        ],
        out_specs=pl.BlockSpec((bm, hd), lambda i: (i, 0)),
        out_shape=jax.ShapeDtypeStruct((n, hd), jnp.bfloat16),
    )(h, w)


def _gru_body(parts_ref, h_ref, wihT_ref, whhT_ref, bih_ref, bhh_ref, o_ref):
    agg = parts_ref[0] + parts_ref[1]
    h = h_ref[...]
    hd = h.shape[1]
    gi = jnp.dot(agg, wihT_ref[...], preferred_element_type=jnp.float32)
    gi = gi + bih_ref[...]
    gh = jnp.dot(h, whhT_ref[...], preferred_element_type=jnp.float32)
    gh = gh + bhh_ref[...]
    r = jax.nn.sigmoid(gi[:, :hd] + gh[:, :hd])
    z = jax.nn.sigmoid(gi[:, hd:2 * hd] + gh[:, hd:2 * hd])
    n = jnp.tanh(gi[:, 2 * hd:] + r * gh[:, 2 * hd:])
    o_ref[...] = (1.0 - z) * n + z * h


def _gru(parts, h, wihT, whhT, bih2, bhh2, bm):
    n, hd = h.shape
    return pl.pallas_call(
        _gru_body,
        grid=(n // bm,),
        in_specs=[
            pl.BlockSpec((NC, bm, hd), lambda i: (0, i, 0)),
            pl.BlockSpec((bm, hd), lambda i: (i, 0)),
            pl.BlockSpec((hd, 3 * hd), lambda i: (0, 0)),
            pl.BlockSpec((hd, 3 * hd), lambda i: (0, 0)),
            pl.BlockSpec((1, 3 * hd), lambda i: (0, 0)),
            pl.BlockSpec((1, 3 * hd), lambda i: (0, 0)),
        ],
        out_specs=pl.BlockSpec((bm, hd), lambda i: (i, 0)),
        out_shape=jax.ShapeDtypeStruct((n, hd), jnp.float32),
    )(parts, h, wihT, whhT, bih2, bhh2)


def _make_sc_scatter(n, hd, nsub):
    vpr = hd // (2 * LANES)
    nchunk = nsub // CW
    z0 = 632
    zlast = n - (NS - 1) * z0

    mesh = plsc.VectorSubcoreMesh(core_axis_name="c", subcore_axis_name="s")

    @functools.partial(
        pl.kernel,
        out_type=jax.ShapeDtypeStruct((NC, n, hd), jnp.float32),
        mesh=mesh,
        compiler_params=pltpu.CompilerParams(use_tc_tiling_on_sc=False),
        scratch_types=[
            pltpu.VMEM((CW, SUB), jnp.int32),
            pltpu.VMEM((CW, SUB), jnp.int32),
            pltpu.VMEM((CW, SUB), jnp.float32),
            [pltpu.VMEM((SUB, hd // 2), jnp.int32)] * 2,
            [pltpu.VMEM((SUB, hd), jnp.float32)] * 2,
            pltpu.VMEM_SHARED((n, hd), jnp.float32),
            [pltpu.SemaphoreType.DMA] * 2,
            [pltpu.SemaphoreType.DMA] * 2,
        ],
    )
    def sc_scatter(m_hbm, src_hbm, dst_hbm, attr_hbm, zeros_hbm, out_hbm,
                   src_v, dst_v, attr_v, grows, srows, agg_sh, sem_g, sem_s):
        cid = lax.axis_index("c")
        sid = lax.axis_index("s")

        def scale(b, kk):
            def group(g, c2):
                a16 = attr_v[kk, pl.ds(g * LANES, LANES)]
                for ei in range(LANES):
                    a = a16[ei]
                    e = g * LANES + ei
                    for j in range(vpr):
                        w16 = grows[b][e, pl.ds(j * LANES, LANES)]
                        lo = lax.bitcast_convert_type(
                            lax.shift_left(w16, jnp.int32(16)), jnp.float32)
                        hi = lax.bitcast_convert_type(
                            lax.bitwise_and(w16, jnp.int32(-65536)),
                            jnp.float32)
                        srows[b][e, pl.ds(2 * j * LANES, LANES)] = lo * a
                        srows[b][e, pl.ds((2 * j + 1) * LANES, LANES)] = \
                            hi * a
                return c2
            lax.fori_loop(0, SUB // LANES, group, 0)

        def gstart(b, kk):
            pltpu.async_copy(m_hbm.at[src_v.at[kk]], grows[b], sem_g[b])

        def gwait(b):
            pltpu.make_async_copy(m_hbm.at[src_v.at[0]], grows[b],
                                  sem_g[b]).wait()

        def sstart(b, kk):
            pltpu.async_copy(srows[b], agg_sh.at[dst_v.at[kk]], sem_s[b],
                             add=True)

        def swait(b):
            pltpu.make_async_copy(srows[b], agg_sh.at[dst_v.at[0]],
                                  sem_s[b]).wait()

        @pl.when(sid < NS - 1)
        def _():
            pltpu.sync_copy(zeros_hbm.at[pl.ds(sid * z0, z0)],
                            agg_sh.at[pl.ds(sid * z0, z0)])

        @pl.when(sid == NS - 1)
        def _():
            pltpu.sync_copy(zeros_hbm.at[pl.ds((NS - 1) * z0, zlast)],
                            agg_sh.at[pl.ds((NS - 1) * z0, zlast)])

        wid = cid * NS + sid
        plsc.subcore_barrier()

        def chunk(c, carry):
            pltpu.sync_copy(src_hbm.at[wid, pl.ds(c * CW, CW)], src_v)
            pltpu.sync_copy(dst_hbm.at[wid, pl.ds(c * CW, CW)], dst_v)
            pltpu.sync_copy(attr_hbm.at[wid, pl.ds(c * CW, CW)], attr_v)

            gstart(0, 0)

            def pair(half, c1):
                k0 = 2 * half
                for p in range(2):
                    b = p
                    nb = (p + 1) % 2
                    gwait(b)

                    @pl.when(2 * half + p > 1)
                    def _(b=b):
                        swait(b)

                    @pl.when(2 * half + p < CW - 1)
                    def _(nb=nb, k0=k0, p=p):
                        gstart(nb, k0 + p + 1)
                    scale(b, k0 + p)
                    sstart(b, k0 + p)
                return c1

            lax.fori_loop(0, CW // 2, pair, 0)
            swait(0)
            swait(1)
            return carry

        lax.fori_loop(0, nchunk, chunk, 0)

        plsc.subcore_barrier()

        @pl.when(sid < NS - 1)
        def _():
            pltpu.sync_copy(agg_sh.at[pl.ds(sid * z0, z0)],
                            out_hbm.at[cid, pl.ds(sid * z0, z0)])

        @pl.when(sid == NS - 1)
        def _():
            pltpu.sync_copy(agg_sh.at[pl.ds((NS - 1) * z0, zlast)],
                            out_hbm.at[cid, pl.ds((NS - 1) * z0, zlast)])

    return sc_scatter


def kernel(x, edge_idx, edge_attr, W, Wih, Whh, bih, bhh):
    n, hd = x.shape
    e = edge_attr.shape[0]
    nl = W.shape[0]
    bm = 1000

    nw = NC * NS
    ept = e // nw
    step = CW * SUB
    ept_pad = -(-ept // step) * step
    nsub = ept_pad // SUB
    npad = ept_pad - ept

    def shard(a, pad_vals):
        a2 = a.reshape(nw, ept)
        if npad:
            a2 = jnp.concatenate([a2, pad_vals], axis=1)
        return a2.reshape(nw, nsub, SUB)

    pad_idx = (jnp.arange(nw * npad, dtype=jnp.int32).reshape(nw, npad)
               * 97) % n if npad else None
    src = shard(edge_idx[0], pad_idx)
    dst = shard(edge_idx[1], pad_idx)
    attr = shard(edge_attr, jnp.zeros((nw, npad), jnp.float32)
                 if npad else None)
    zeros = jnp.zeros((n, hd), jnp.float32)

    perm = _feature_perm(hd)
    Wp = W[:, :, perm]
    wihT = jnp.swapaxes(Wih, 1, 2)
    whhT = jnp.swapaxes(Whh, 1, 2)
    bih2 = bih.reshape(nl, 1, -1)
    bhh2 = bhh.reshape(nl, 1, -1)

    sc_scatter = _make_sc_scatter(n, hd, nsub)

    h = x
    for l in range(nl):
        m_bf = _matmul(h, Wp[l], bm)
        m_pk = lax.bitcast_convert_type(
            m_bf.reshape(n, hd // 2, 2), jnp.int32)
        parts = sc_scatter(m_pk, src, dst, attr, zeros)
        h = _gru(parts, h, wihT[l], whhT[l], bih2[l], bhh2[l], bm)
    return h


# speedup vs baseline: 1.6128x; 1.6128x over previous
import functools

import jax
import jax.numpy as jnp
from jax import lax
from jax.experimental import pallas as pl
from jax.experimental.pallas import tpu as pltpu
from jax.experimental.pallas import tpu_sc as plsc

LANES = 16
SUB = 80
CW = 16
NBUF = 4
NC = 2
NS = 16


def _mm_body(h_ref, w_ref, o_ref):
    o_ref[...] = jnp.dot(h_ref[...], w_ref[...],
                         preferred_element_type=jnp.float32)


def _matmul(h, w, bm):
    n, hd = h.shape
    return pl.pallas_call(
        _mm_body,
        grid=(n // bm,),
        in_specs=[
            pl.BlockSpec((bm, hd), lambda i: (i, 0)),
            pl.BlockSpec((hd, hd), lambda i: (0, 0)),
        ],
        out_specs=pl.BlockSpec((bm, hd), lambda i: (i, 0)),
        out_shape=jax.ShapeDtypeStruct((n, hd), jnp.float32),
    )(h, w)


def _gru_core(parts_ref, h_ref, wihT_ref, whhT_ref, bih_ref, bhh_ref):
    agg = parts_ref[0] + parts_ref[1]
    h = h_ref[...]
    hd = h.shape[1]
    gi = jnp.dot(agg, wihT_ref[...], preferred_element_type=jnp.float32)
    gi = gi + bih_ref[...]
    gh = jnp.dot(h, whhT_ref[...], preferred_element_type=jnp.float32)
    gh = gh + bhh_ref[...]
    r = jax.nn.sigmoid(gi[:, :hd] + gh[:, :hd])
    z = jax.nn.sigmoid(gi[:, hd:2 * hd] + gh[:, hd:2 * hd])
    nn = jnp.tanh(gi[:, 2 * hd:] + r * gh[:, 2 * hd:])
    return (1.0 - z) * nn + z * h


def _gru_body(parts_ref, h_ref, wihT_ref, whhT_ref, bih_ref, bhh_ref, o_ref):
    o_ref[...] = _gru_core(parts_ref, h_ref, wihT_ref, whhT_ref,
                           bih_ref, bhh_ref)


def _gru_mm_body(parts_ref, h_ref, wihT_ref, whhT_ref, bih_ref, bhh_ref,
                 wnext_ref, o_ref, m_ref):
    hn = _gru_core(parts_ref, h_ref, wihT_ref, whhT_ref, bih_ref, bhh_ref)
    o_ref[...] = hn
    m_ref[...] = jnp.dot(hn, wnext_ref[...],
                         preferred_element_type=jnp.float32)


def _gru(parts, h, wihT, whhT, bih2, bhh2, bm, wnext=None):
    n, hd = h.shape
    specs = [
        pl.BlockSpec((NC, bm, hd), lambda i: (0, i, 0)),
        pl.BlockSpec((bm, hd), lambda i: (i, 0)),
        pl.BlockSpec((hd, 3 * hd), lambda i: (0, 0)),
        pl.BlockSpec((hd, 3 * hd), lambda i: (0, 0)),
        pl.BlockSpec((1, 3 * hd), lambda i: (0, 0)),
        pl.BlockSpec((1, 3 * hd), lambda i: (0, 0)),
    ]
    out_spec = pl.BlockSpec((bm, hd), lambda i: (i, 0))
    out_shape = jax.ShapeDtypeStruct((n, hd), jnp.float32)
    if wnext is None:
        return pl.pallas_call(
            _gru_body, grid=(n // bm,), in_specs=specs,
            out_specs=out_spec, out_shape=out_shape,
        )(parts, h, wihT, whhT, bih2, bhh2)
    return pl.pallas_call(
        _gru_mm_body, grid=(n // bm,),
        in_specs=specs + [pl.BlockSpec((hd, hd), lambda i: (0, 0))],
        out_specs=(out_spec, out_spec), out_shape=(out_shape, out_shape),
    )(parts, h, wihT, whhT, bih2, bhh2, wnext)


def _make_sc_scatter(n, hd, nsub):
    vpr = hd // LANES
    nchunk = nsub // CW
    z0 = 632
    zlast = n - (NS - 1) * z0

    mesh = plsc.VectorSubcoreMesh(core_axis_name="c", subcore_axis_name="s")

    @functools.partial(
        pl.kernel,
        out_type=jax.ShapeDtypeStruct((NC, n, hd), jnp.float32),
        mesh=mesh,
        scratch_types=[
            pltpu.VMEM((CW, SUB), jnp.int32),
            pltpu.VMEM((CW, SUB), jnp.int32),
            pltpu.VMEM((CW, SUB), jnp.float32),
            [pltpu.VMEM((SUB, hd), jnp.float32)] * NBUF,
            pltpu.VMEM_SHARED((n, hd), jnp.float32),
            [pltpu.SemaphoreType.DMA] * NBUF,
            [pltpu.SemaphoreType.DMA] * NBUF,
        ],
    )
    def sc_scatter(m_hbm, src_hbm, dst_hbm, attr_hbm, zeros_hbm, out_hbm,
                   src_v, dst_v, attr_v, rows, agg_sh, sem_g, sem_s):
        cid = lax.axis_index("c")
        sid = lax.axis_index("s")

        def scale(b, kk):
            def group(g, c2):
                a16 = attr_v[kk, pl.ds(g * LANES, LANES)]
                for ei in range(LANES):
                    a = a16[ei]
                    for j in range(vpr):
                        sl = pl.ds(j * LANES, LANES)
                        rows[b][g * LANES + ei, sl] = \
                            rows[b][g * LANES + ei, sl] * a
                return c2
            lax.fori_loop(0, SUB // LANES, group, 0)

        def gstart(b, kk):
            pltpu.async_copy(m_hbm.at[src_v.at[kk]], rows[b], sem_g[b])

        def gwait(b):
            pltpu.make_async_copy(m_hbm.at[src_v.at[0]], rows[b],
                                  sem_g[b]).wait()

        def sstart(b, kk):
            pltpu.async_copy(rows[b], agg_sh.at[dst_v.at[kk]], sem_s[b],
                             add=True)

        def swait(b):
            pltpu.make_async_copy(rows[b], agg_sh.at[dst_v.at[0]],
                                  sem_s[b]).wait()

        @pl.when(sid < NS - 1)
        def _():
            pltpu.sync_copy(zeros_hbm.at[pl.ds(sid * z0, z0)],
                            agg_sh.at[pl.ds(sid * z0, z0)])

        @pl.when(sid == NS - 1)
        def _():
            pltpu.sync_copy(zeros_hbm.at[pl.ds((NS - 1) * z0, zlast)],
                            agg_sh.at[pl.ds((NS - 1) * z0, zlast)])

        wid = cid * NS + sid
        plsc.subcore_barrier()

        def chunk(c, carry):
            pltpu.sync_copy(src_hbm.at[wid, pl.ds(c * CW, CW)], src_v)
            pltpu.sync_copy(dst_hbm.at[wid, pl.ds(c * CW, CW)], dst_v)
            pltpu.sync_copy(attr_hbm.at[wid, pl.ds(c * CW, CW)], attr_v)

            gstart(0, 0)

            def quad(q, c1):
                k0 = 4 * q
                for p in range(NBUF):
                    b = p
                    nb = (p + 1) % NBUF
                    gwait(b)
                    if p < NBUF - 1:
                        @pl.when(q > 0)
                        def _(nb=nb):
                            swait(nb)

                        gstart(nb, k0 + p + 1)
                    else:
                        swait(nb)

                        @pl.when(q < CW // NBUF - 1)
                        def _(nb=nb, k0=k0):
                            gstart(nb, k0 + NBUF)
                    scale(b, k0 + p)
                    sstart(b, k0 + p)
                return c1

            lax.fori_loop(0, CW // NBUF, quad, 0)
            for b in range(1, NBUF):
                swait(b)
            return carry

        lax.fori_loop(0, nchunk, chunk, 0)

        plsc.subcore_barrier()

        @pl.when(sid < NS - 1)
        def _():
            pltpu.sync_copy(agg_sh.at[pl.ds(sid * z0, z0)],
                            out_hbm.at[cid, pl.ds(sid * z0, z0)])

        @pl.when(sid == NS - 1)
        def _():
            pltpu.sync_copy(agg_sh.at[pl.ds((NS - 1) * z0, zlast)],
                            out_hbm.at[cid, pl.ds((NS - 1) * z0, zlast)])

    return sc_scatter


def kernel(x, edge_idx, edge_attr, W, Wih, Whh, bih, bhh):
    n, hd = x.shape
    e = edge_attr.shape[0]
    nl = W.shape[0]
    bm = 1000

    nw = NC * NS
    ept = e // nw
    step = CW * SUB
    ept_pad = -(-ept // step) * step
    nsub = ept_pad // SUB
    npad = ept_pad - ept

    def shard(a, pad_vals):
        a2 = a.reshape(nw, ept)
        if npad:
            a2 = jnp.concatenate([a2, pad_vals], axis=1)
        return a2.reshape(nw, nsub, SUB)

    pad_idx = (jnp.arange(nw * npad, dtype=jnp.int32).reshape(nw, npad)
               * 97) % n if npad else None
    src = shard(edge_idx[0], pad_idx)
    dst = shard(edge_idx[1], pad_idx)
    attr = shard(edge_attr, jnp.zeros((nw, npad), jnp.float32)
                 if npad else None)
    zeros = jnp.zeros((n, hd), jnp.float32)

    wihT = jnp.swapaxes(Wih, 1, 2)
    whhT = jnp.swapaxes(Whh, 1, 2)
    bih2 = bih.reshape(nl, 1, -1)
    bhh2 = bhh.reshape(nl, 1, -1)

    sc_scatter = _make_sc_scatter(n, hd, nsub)

    h = x
    m = _matmul(h, W[0], bm)
    for l in range(nl):
        parts = sc_scatter(m, src, dst, attr, zeros)
        if l + 1 < nl:
            h, m = _gru(parts, h, wihT[l], whhT[l], bih2[l], bhh2[l], bm,
                        wnext=W[l + 1])
        else:
            h = _gru(parts, h, wihT[l], whhT[l], bih2[l], bhh2[l], bm)
    return h


# speedup vs baseline: 1.7382x; 1.0778x over previous
import functools

import jax
import jax.numpy as jnp
from jax import lax
from jax.experimental import pallas as pl
from jax.experimental.pallas import tpu as pltpu
from jax.experimental.pallas import tpu_sc as plsc

LANES = 16
SUB = 80
GW = 4
NBUF = 4
NC = 2
NS = 16


def _mm_body(h_ref, w_ref, o_ref):
    o_ref[...] = jnp.dot(h_ref[...], w_ref[...],
                         preferred_element_type=jnp.float32)


def _matmul(h, w, bm):
    n, hd = h.shape
    return pl.pallas_call(
        _mm_body,
        grid=(n // bm,),
        in_specs=[
            pl.BlockSpec((bm, hd), lambda i: (i, 0)),
            pl.BlockSpec((hd, hd), lambda i: (0, 0)),
        ],
        out_specs=pl.BlockSpec((bm, hd), lambda i: (i, 0)),
        out_shape=jax.ShapeDtypeStruct((n, hd), jnp.float32),
    )(h, w)


def _gru_core(parts_ref, h_ref, wihT_ref, whhT_ref, bih_ref, bhh_ref):
    agg = parts_ref[0] + parts_ref[1]
    h = h_ref[...]
    hd = h.shape[1]
    gi = jnp.dot(agg, wihT_ref[...], preferred_element_type=jnp.float32)
    gi = gi + bih_ref[...]
    gh = jnp.dot(h, whhT_ref[...], preferred_element_type=jnp.float32)
    gh = gh + bhh_ref[...]
    r = jax.nn.sigmoid(gi[:, :hd] + gh[:, :hd])
    z = jax.nn.sigmoid(gi[:, hd:2 * hd] + gh[:, hd:2 * hd])
    nn = jnp.tanh(gi[:, 2 * hd:] + r * gh[:, 2 * hd:])
    return (1.0 - z) * nn + z * h


def _gru_body(parts_ref, h_ref, wihT_ref, whhT_ref, bih_ref, bhh_ref, o_ref):
    o_ref[...] = _gru_core(parts_ref, h_ref, wihT_ref, whhT_ref,
                           bih_ref, bhh_ref)


def _gru_mm_body(parts_ref, h_ref, wihT_ref, whhT_ref, bih_ref, bhh_ref,
                 wnext_ref, o_ref, m_ref):
    hn = _gru_core(parts_ref, h_ref, wihT_ref, whhT_ref, bih_ref, bhh_ref)
    o_ref[...] = hn
    m_ref[...] = jnp.dot(hn, wnext_ref[...],
                         preferred_element_type=jnp.float32)


def _gru(parts, h, wihT, whhT, bih2, bhh2, bm, wnext=None):
    n, hd = h.shape
    specs = [
        pl.BlockSpec((NC, bm, hd), lambda i: (0, i, 0)),
        pl.BlockSpec((bm, hd), lambda i: (i, 0)),
        pl.BlockSpec((hd, 3 * hd), lambda i: (0, 0)),
        pl.BlockSpec((hd, 3 * hd), lambda i: (0, 0)),
        pl.BlockSpec((1, 3 * hd), lambda i: (0, 0)),
        pl.BlockSpec((1, 3 * hd), lambda i: (0, 0)),
    ]
    out_spec = pl.BlockSpec((bm, hd), lambda i: (i, 0))
    out_shape = jax.ShapeDtypeStruct((n, hd), jnp.float32)
    if wnext is None:
        return pl.pallas_call(
            _gru_body, grid=(n // bm,), in_specs=specs,
            out_specs=out_spec, out_shape=out_shape,
        )(parts, h, wihT, whhT, bih2, bhh2)
    return pl.pallas_call(
        _gru_mm_body, grid=(n // bm,),
        in_specs=specs + [pl.BlockSpec((hd, hd), lambda i: (0, 0))],
        out_specs=(out_spec, out_spec), out_shape=(out_shape, out_shape),
    )(parts, h, wihT, whhT, bih2, bhh2, wnext)


def _make_sc_scatter(n, hd, nsub):
    vpr = hd // LANES
    ngrp = nsub // GW
    z0 = 632
    zlast = n - (NS - 1) * z0

    mesh = plsc.VectorSubcoreMesh(core_axis_name="c", subcore_axis_name="s")

    @functools.partial(
        pl.kernel,
        out_type=jax.ShapeDtypeStruct((NC, n, hd), jnp.float32),
        mesh=mesh,
        scratch_types=[
            [pltpu.VMEM((GW, SUB), jnp.int32)] * 2,
            [pltpu.VMEM((GW, SUB), jnp.int32)] * 2,
            [pltpu.VMEM((GW, SUB), jnp.float32)] * 2,
            [pltpu.VMEM((SUB, hd), jnp.float32)] * NBUF,
            pltpu.VMEM_SHARED((n, hd), jnp.float32),
            [pltpu.SemaphoreType.DMA] * NBUF,
            [pltpu.SemaphoreType.DMA] * NBUF,
            [pltpu.SemaphoreType.DMA] * 2,
        ],
    )
    def sc_scatter(m_hbm, src_hbm, dst_hbm, attr_hbm, zeros_hbm, out_hbm,
                   src_v, dst_v, attr_v, rows, agg_sh, sem_g, sem_s, sem_i):
        cid = lax.axis_index("c")
        sid = lax.axis_index("s")
        wid = cid * NS + sid

        def scale(b, par, kk):
            def group(g, c2):
                a16 = attr_v[par][kk, pl.ds(g * LANES, LANES)]
                for ei in range(LANES):
                    a = a16[ei]
                    for j in range(vpr):
                        sl = pl.ds(j * LANES, LANES)
                        rows[b][g * LANES + ei, sl] = \
                            rows[b][g * LANES + ei, sl] * a
                return c2
            lax.fori_loop(0, SUB // LANES, group, 0)

        def gstart(b, par, kk):
            pltpu.async_copy(m_hbm.at[src_v[par].at[kk]], rows[b], sem_g[b])

        def gwait(b):
            pltpu.make_async_copy(m_hbm.at[src_v[0].at[0]], rows[b],
                                  sem_g[b]).wait()

        def sstart(b, par, kk):
            pltpu.async_copy(rows[b], agg_sh.at[dst_v[par].at[kk]],
                             sem_s[b], add=True)

        def swait(b):
            pltpu.make_async_copy(rows[b], agg_sh.at[dst_v[0].at[0]],
                                  sem_s[b]).wait()

        def istart(par, g):
            pltpu.async_copy(src_hbm.at[wid, g], src_v[par], sem_i[par])
            pltpu.async_copy(dst_hbm.at[wid, g], dst_v[par], sem_i[par])
            pltpu.async_copy(attr_hbm.at[wid, g], attr_v[par], sem_i[par])

        def iwait(par):
            pltpu.make_async_copy(src_hbm.at[wid, 0], src_v[par],
                                  sem_i[par]).wait()
            pltpu.make_async_copy(dst_hbm.at[wid, 0], dst_v[par],
                                  sem_i[par]).wait()
            pltpu.make_async_copy(attr_hbm.at[wid, 0], attr_v[par],
                                  sem_i[par]).wait()

        @pl.when(sid < NS - 1)
        def _():
            pltpu.sync_copy(zeros_hbm.at[pl.ds(sid * z0, z0)],
                            agg_sh.at[pl.ds(sid * z0, z0)])

        @pl.when(sid == NS - 1)
        def _():
            pltpu.sync_copy(zeros_hbm.at[pl.ds((NS - 1) * z0, zlast)],
                            agg_sh.at[pl.ds((NS - 1) * z0, zlast)])

        istart(0, 0)
        iwait(0)
        plsc.subcore_barrier()
        gstart(0, 0, 0)

        def pairbody(i, carry):
            for par in range(2):
                g = 2 * i + par
                w0 = g * GW
                for p in range(GW):
                    b = p
                    nb = (p + 1) % NBUF
                    w = w0 + p
                    gwait(b)

                    @pl.when(w >= 3)
                    def _(b=b, nb=nb):
                        swait(nb)

                    if p == 2:
                        @pl.when(g < ngrp - 1)
                        def _(par=par, g=g):
                            istart(1 - par, g + 1)
                    if p < GW - 1:
                        gstart(nb, par, p + 1)
                    else:
                        @pl.when(g < ngrp - 1)
                        def _(nb=nb, par=par):
                            iwait(1 - par)
                            gstart(nb, 1 - par, 0)
                    scale(b, par, p)
                    sstart(b, par, p)
            return carry

        lax.fori_loop(0, ngrp // 2, pairbody, 0)
        for b in range(1, NBUF):
            swait(b)

        plsc.subcore_barrier()

        @pl.when(sid < NS - 1)
        def _():
            pltpu.sync_copy(agg_sh.at[pl.ds(sid * z0, z0)],
                            out_hbm.at[cid, pl.ds(sid * z0, z0)])

        @pl.when(sid == NS - 1)
        def _():
            pltpu.sync_copy(agg_sh.at[pl.ds((NS - 1) * z0, zlast)],
                            out_hbm.at[cid, pl.ds((NS - 1) * z0, zlast)])

    return sc_scatter


def kernel(x, edge_idx, edge_attr, W, Wih, Whh, bih, bhh):
    n, hd = x.shape
    e = edge_attr.shape[0]
    nl = W.shape[0]
    bm = 1000

    nw = NC * NS
    ept = e // nw
    step = 2 * GW * SUB
    ept_pad = -(-ept // step) * step
    nsub = ept_pad // SUB
    npad = ept_pad - ept

    def shard(a, pad_vals):
        a2 = a.reshape(nw, ept)
        if npad:
            a2 = jnp.concatenate([a2, pad_vals], axis=1)
        return a2.reshape(nw, nsub // GW, GW, SUB)

    pad_idx = (jnp.arange(nw * npad, dtype=jnp.int32).reshape(nw, npad)
               * 97) % n if npad else None
    src = shard(edge_idx[0], pad_idx)
    dst = shard(edge_idx[1], pad_idx)
    attr = shard(edge_attr, jnp.zeros((nw, npad), jnp.float32)
                 if npad else None)
    zeros = jnp.zeros((n, hd), jnp.float32)

    wihT = jnp.swapaxes(Wih, 1, 2)
    whhT = jnp.swapaxes(Whh, 1, 2)
    bih2 = bih.reshape(nl, 1, -1)
    bhh2 = bhh.reshape(nl, 1, -1)

    sc_scatter = _make_sc_scatter(n, hd, nsub)

    h = x
    m = _matmul(h, W[0], bm)
    for l in range(nl):
        parts = sc_scatter(m, src, dst, attr, zeros)
        if l + 1 < nl:
            h, m = _gru(parts, h, wihT[l], whhT[l], bih2[l], bhh2[l], bm,
                        wnext=W[l + 1])
        else:
            h = _gru(parts, h, wihT[l], whhT[l], bih2[l], bhh2[l], bm)
    return h


# speedup vs baseline: 1.7686x; 1.0175x over previous
import functools

import jax
import jax.numpy as jnp
from jax import lax
from jax.experimental import pallas as pl
from jax.experimental.pallas import tpu as pltpu
from jax.experimental.pallas import tpu_sc as plsc

LANES = 16
SUB = 80
GW = 4
NBUF = 4
NC = 2
NS = 16


def _mm_body(h_ref, w_ref, o_ref):
    o_ref[...] = jnp.dot(h_ref[...], w_ref[...],
                         preferred_element_type=jnp.float32)


def _matmul(h, w, bm):
    n, hd = h.shape
    return pl.pallas_call(
        _mm_body,
        grid=(n // bm,),
        in_specs=[
            pl.BlockSpec((bm, hd), lambda i: (i, 0)),
            pl.BlockSpec((hd, hd), lambda i: (0, 0)),
        ],
        out_specs=pl.BlockSpec((bm, hd), lambda i: (i, 0)),
        out_shape=jax.ShapeDtypeStruct((n, hd), jnp.float32),
    )(h, w)


def _gru_core(parts_ref, h_ref, wihT_ref, whhT_ref, bih_ref, bhh_ref):
    agg = parts_ref[0] + parts_ref[1]
    h = h_ref[...]
    hd = h.shape[1]
    gi = jnp.dot(agg, wihT_ref[...], preferred_element_type=jnp.float32)
    gi = gi + bih_ref[...]
    gh = jnp.dot(h, whhT_ref[...], preferred_element_type=jnp.float32)
    gh = gh + bhh_ref[...]
    r = jax.nn.sigmoid(gi[:, :hd] + gh[:, :hd])
    z = jax.nn.sigmoid(gi[:, hd:2 * hd] + gh[:, hd:2 * hd])
    nn = jnp.tanh(gi[:, 2 * hd:] + r * gh[:, 2 * hd:])
    return (1.0 - z) * nn + z * h


def _gru_body(parts_ref, h_ref, wihT_ref, whhT_ref, bih_ref, bhh_ref, o_ref):
    o_ref[...] = _gru_core(parts_ref, h_ref, wihT_ref, whhT_ref,
                           bih_ref, bhh_ref)


def _gru_mm_body(parts_ref, h_ref, wihT_ref, whhT_ref, bih_ref, bhh_ref,
                 wnext_ref, o_ref, m_ref):
    hn = _gru_core(parts_ref, h_ref, wihT_ref, whhT_ref, bih_ref, bhh_ref)
    o_ref[...] = hn
    m_ref[...] = jnp.dot(hn, wnext_ref[...],
                         preferred_element_type=jnp.float32)


def _gru(parts, h, wihT, whhT, bih2, bhh2, bm, wnext=None):
    n, hd = h.shape
    specs = [
        pl.BlockSpec((NC, bm, hd), lambda i: (0, i, 0)),
        pl.BlockSpec((bm, hd), lambda i: (i, 0)),
        pl.BlockSpec((hd, 3 * hd), lambda i: (0, 0)),
        pl.BlockSpec((hd, 3 * hd), lambda i: (0, 0)),
        pl.BlockSpec((1, 3 * hd), lambda i: (0, 0)),
        pl.BlockSpec((1, 3 * hd), lambda i: (0, 0)),
    ]
    out_spec = pl.BlockSpec((bm, hd), lambda i: (i, 0))
    out_shape = jax.ShapeDtypeStruct((n, hd), jnp.float32)
    if wnext is None:
        return pl.pallas_call(
            _gru_body, grid=(n // bm,), in_specs=specs,
            out_specs=out_spec, out_shape=out_shape,
        )(parts, h, wihT, whhT, bih2, bhh2)
    return pl.pallas_call(
        _gru_mm_body, grid=(n // bm,),
        in_specs=specs + [pl.BlockSpec((hd, hd), lambda i: (0, 0))],
        out_specs=(out_spec, out_spec), out_shape=(out_shape, out_shape),
    )(parts, h, wihT, whhT, bih2, bhh2, wnext)


def _make_sc_scatter(n, hd, nsub):
    vpr = hd // LANES
    ngrp = nsub // GW
    z0 = 632
    zlast = n - (NS - 1) * z0

    mesh = plsc.VectorSubcoreMesh(core_axis_name="c", subcore_axis_name="s")

    @functools.partial(
        pl.kernel,
        out_type=jax.ShapeDtypeStruct((NC, n, hd), jnp.float32),
        mesh=mesh,
        scratch_types=[
            [pltpu.VMEM((GW, SUB), jnp.int32)] * 2,
            [pltpu.VMEM((GW, SUB), jnp.int32)] * 2,
            [pltpu.VMEM((GW, SUB), jnp.float32)] * 2,
            [pltpu.VMEM((SUB, hd), jnp.float32)] * NBUF,
            pltpu.VMEM_SHARED((n, hd), jnp.float32),
            [pltpu.SemaphoreType.DMA] * NBUF,
            [pltpu.SemaphoreType.DMA] * NBUF,
            [pltpu.SemaphoreType.DMA] * 2,
        ],
    )
    def sc_scatter(m_hbm, src_hbm, dst_hbm, attr_hbm, zeros_hbm, out_hbm,
                   src_v, dst_v, attr_v, rows, agg_sh, sem_g, sem_s, sem_i):
        cid = lax.axis_index("c")
        sid = lax.axis_index("s")
        wid = cid * NS + sid

        def scale(b, par, kk):
            def group(g, c2):
                a16 = attr_v[par][kk, pl.ds(g * LANES, LANES)]
                for ei in range(LANES):
                    a = a16[ei]
                    for j in range(vpr):
                        sl = pl.ds(j * LANES, LANES)
                        rows[b][g * LANES + ei, sl] = \
                            rows[b][g * LANES + ei, sl] * a
                return c2
            lax.fori_loop(0, SUB // LANES, group, 0)

        def gstart(b, par, kk):
            pltpu.async_copy(m_hbm.at[src_v[par].at[kk]], rows[b], sem_g[b])

        def gwait(b):
            pltpu.make_async_copy(m_hbm.at[src_v[0].at[0]], rows[b],
                                  sem_g[b]).wait()

        def sstart(b, par, kk):
            pltpu.async_copy(rows[b], agg_sh.at[dst_v[par].at[kk]],
                             sem_s[b], add=True)

        def swait(b):
            pltpu.make_async_copy(rows[b], agg_sh.at[dst_v[0].at[0]],
                                  sem_s[b]).wait()

        def istart(par, g):
            pltpu.async_copy(src_hbm.at[wid, g], src_v[par], sem_i[par])
            pltpu.async_copy(dst_hbm.at[wid, g], dst_v[par], sem_i[par])
            pltpu.async_copy(attr_hbm.at[wid, g], attr_v[par], sem_i[par])

        def iwait(par):
            pltpu.make_async_copy(src_hbm.at[wid, 0], src_v[par],
                                  sem_i[par]).wait()
            pltpu.make_async_copy(dst_hbm.at[wid, 0], dst_v[par],
                                  sem_i[par]).wait()
            pltpu.make_async_copy(attr_hbm.at[wid, 0], attr_v[par],
                                  sem_i[par]).wait()

        @pl.when(sid < NS - 1)
        def _():
            pltpu.sync_copy(zeros_hbm.at[pl.ds(sid * z0, z0)],
                            agg_sh.at[pl.ds(sid * z0, z0)])

        @pl.when(sid == NS - 1)
        def _():
            pltpu.sync_copy(zeros_hbm.at[pl.ds((NS - 1) * z0, zlast)],
                            agg_sh.at[pl.ds((NS - 1) * z0, zlast)])

        istart(0, 0)
        iwait(0)
        plsc.subcore_barrier()
        gstart(0, 0, 0)

        def pairbody(i, carry):
            for par in range(2):
                g = 2 * i + par
                w0 = g * GW
                for p in range(GW):
                    b = p
                    nb = (p + 1) % NBUF
                    w = w0 + p
                    gwait(b)

                    @pl.when(w >= 3)
                    def _(b=b, nb=nb):
                        swait(nb)

                    if p == 2:
                        @pl.when(g < ngrp - 1)
                        def _(par=par, g=g):
                            istart(1 - par, g + 1)
                    if p < GW - 1:
                        gstart(nb, par, p + 1)
                    else:
                        @pl.when(g < ngrp - 1)
                        def _(nb=nb, par=par):
                            iwait(1 - par)
                            gstart(nb, 1 - par, 0)
                    scale(b, par, p)
                    sstart(b, par, p)
            return carry

        lax.fori_loop(0, ngrp // 2, pairbody, 0)
        for b in range(1, NBUF):
            swait(b)

        plsc.subcore_barrier()

        @pl.when(sid < NS - 1)
        def _():
            pltpu.sync_copy(agg_sh.at[pl.ds(sid * z0, z0)],
                            out_hbm.at[cid, pl.ds(sid * z0, z0)])

        @pl.when(sid == NS - 1)
        def _():
            pltpu.sync_copy(agg_sh.at[pl.ds((NS - 1) * z0, zlast)],
                            out_hbm.at[cid, pl.ds((NS - 1) * z0, zlast)])

    return sc_scatter


def kernel(x, edge_idx, edge_attr, W, Wih, Whh, bih, bhh):
    n, hd = x.shape
    e = edge_attr.shape[0]
    nl = W.shape[0]
    bm = 2000

    nw = NC * NS
    ept = e // nw
    step = 2 * GW * SUB
    ept_pad = -(-ept // step) * step
    nsub = ept_pad // SUB
    npad = ept_pad - ept

    def shard(a, pad_vals):
        a2 = a.reshape(nw, ept)
        if npad:
            a2 = jnp.concatenate([a2, pad_vals], axis=1)
        return a2.reshape(nw, nsub // GW, GW, SUB)

    pad_idx = (jnp.arange(nw * npad, dtype=jnp.int32).reshape(nw, npad)
               * 97) % n if npad else None
    src = shard(edge_idx[0], pad_idx)
    dst = shard(edge_idx[1], pad_idx)
    attr = shard(edge_attr, jnp.zeros((nw, npad), jnp.float32)
                 if npad else None)
    zeros = jnp.zeros((n, hd), jnp.float32)

    wihT = jnp.swapaxes(Wih, 1, 2)
    whhT = jnp.swapaxes(Whh, 1, 2)
    bih2 = bih.reshape(nl, 1, -1)
    bhh2 = bhh.reshape(nl, 1, -1)

    sc_scatter = _make_sc_scatter(n, hd, nsub)

    h = x
    m = _matmul(h, W[0], bm)
    for l in range(nl):
        parts = sc_scatter(m, src, dst, attr, zeros)
        if l + 1 < nl:
            h, m = _gru(parts, h, wihT[l], whhT[l], bih2[l], bhh2[l], bm,
                        wnext=W[l + 1])
        else:
            h = _gru(parts, h, wihT[l], whhT[l], bih2[l], bhh2[l], bm)
    return h
